# baseline (speedup 1.0000x reference)
import jax
import jax.numpy as jnp
from jax.experimental import pallas as pl

_TI = 11


def _quant_pair(ref, bb, idx):
    a = jnp.clip(jnp.round(ref[bb, :, idx] * 255.0), 0.0, 255.0)
    return a[:, 0:1, :] + a[:, 1:2, :]


def _body(xref, wr, br, outr):
    W = 112
    ti0 = _TI * pl.program_id(0)

    w = wr[...]
    m00 = w[0] + w[1] + w[2] + w[3]
    m01 = w[1] - w[2] - w[3]
    m10 = w[1] + w[2] - w[3] - w[4]
    m11 = w[1] - w[2] + w[3] - w[4]
    m4 = jnp.concatenate([m00, m01, m10, m11], axis=1)

    bias = br[...]
    lane = jax.lax.broadcasted_iota(jnp.int32, (64, 128), 1)
    even = lane % 2 == 0
    zpad = jnp.zeros((32, 16), jnp.float32)

    for bb in range(2):
        p = [_quant_pair(xref, bb, ti0 + j) for j in range(_TI + 1)]
        cols = []
        for j in range(_TI):
            v = jnp.reshape(p[j] + p[j + 1], (32, W))
            h1 = v + jnp.concatenate([v[:, 1:], v[:, :1]], axis=1)
            h2 = h1 + jnp.concatenate([h1[:, 2:], h1[:, :2]], axis=1)
            u = jnp.minimum(h2, 255.0)
            cols.append(jnp.concatenate([u, zpad], axis=1))
        ub = jnp.concatenate(cols, axis=1)

        z = jax.lax.dot_general(m4, ub, (((0,), (0,)), ((), ())),
                                preferred_element_type=jnp.float32)

        for j in range(_TI):
            zj = z[:, j * 128:(j + 1) * 128]
            for prow in range(2):
                ev = zj[prow * 128:prow * 128 + 64]
                od = zj[prow * 128 + 64:prow * 128 + 128]
                odr = jnp.concatenate([od[:, -1:], od[:, :-1]], axis=1)
                row = jnp.where(even, ev, odr) + bias
                outr[bb, :, j, prow, :] = row[:, :110]


def kernel(x, pcilt, bias):
    b, c, h, w = x.shape
    th = (h - 4) // 2 + 1
    tw = (w - 4) // 2 + 1
    o = pcilt.shape[0]

    x5 = x.reshape(b, c, h // 2, 2, w)
    qw = pcilt[:, :, :, :, 1]
    w5 = jnp.stack([qw[:, :, 0, 0], qw[:, :, 1, 1], qw[:, :, 1, 2],
                    qw[:, :, 2, 2], qw[:, :, 3, 1]], axis=0)
    w5 = jnp.transpose(w5, (0, 2, 1))
    bias2 = bias.reshape(o, 1)

    out = pl.pallas_call(
        _body,
        grid=(th // _TI,),
        in_specs=[
            pl.BlockSpec((b, c, h // 2, 2, w), lambda k: (0, 0, 0, 0, 0)),
            pl.BlockSpec((5, c, o), lambda k: (0, 0, 0)),
            pl.BlockSpec((o, 1), lambda k: (0, 0)),
        ],
        out_specs=pl.BlockSpec((b, o, _TI, 2, 2 * tw), lambda k: (0, 0, k, 0, 0)),
        out_shape=jax.ShapeDtypeStruct((b, o, th, 2, 2 * tw), jnp.float32),
    )(x5, w5, bias2)

    return out.reshape(b, o, 2 * th, 2 * tw)


# speedup vs baseline: 1.0014x; 1.0014x over previous
import jax
import jax.numpy as jnp
from jax.experimental import pallas as pl
from jax.experimental.pallas import tpu as pltpu

_TI = 11


def _quant_pair(ref, bb, idx):
    a = jnp.clip(jnp.round(ref[bb, :, idx] * 255.0), 0.0, 255.0)
    return a[:, 0:1, :] + a[:, 1:2, :]


def _body(xref, wr, br, outr):
    W = 112
    ti0 = _TI * pl.program_id(0)

    w = wr[...]
    m00 = w[0] + w[1] + w[2] + w[3]
    m01 = w[1] - w[2] - w[3]
    m10 = w[1] + w[2] - w[3] - w[4]
    m11 = w[1] - w[2] + w[3] - w[4]
    m4 = jnp.concatenate([m00, m01, m10, m11], axis=1)

    bias = br[...]
    lane = jax.lax.broadcasted_iota(jnp.int32, (64, 128), 1)
    even = lane % 2 == 0
    zpad = jnp.zeros((32, 16), jnp.float32)

    for bb in range(2):
        p = [_quant_pair(xref, bb, ti0 + j) for j in range(_TI + 1)]
        cols = []
        for j in range(_TI):
            v = jnp.reshape(p[j] + p[j + 1], (32, W))
            h1 = v + jnp.concatenate([v[:, 1:], v[:, :1]], axis=1)
            h2 = h1 + jnp.concatenate([h1[:, 2:], h1[:, :2]], axis=1)
            u = jnp.minimum(h2, 255.0)
            cols.append(jnp.concatenate([u, zpad], axis=1))
        ub = jnp.concatenate(cols, axis=1)

        z = jax.lax.dot_general(m4, ub, (((0,), (0,)), ((), ())),
                                preferred_element_type=jnp.float32)

        for j in range(_TI):
            zj = z[:, j * 128:(j + 1) * 128]
            for prow in range(2):
                ev = zj[prow * 128:prow * 128 + 64]
                od = zj[prow * 128 + 64:prow * 128 + 128]
                odr = jnp.concatenate([od[:, -1:], od[:, :-1]], axis=1)
                row = jnp.where(even, ev, odr) + bias
                outr[bb, :, j, prow, :] = row[:, :110]


def kernel(x, pcilt, bias):
    b, c, h, w = x.shape
    th = (h - 4) // 2 + 1
    tw = (w - 4) // 2 + 1
    o = pcilt.shape[0]

    x5 = x.reshape(b, c, h // 2, 2, w)
    qw = pcilt[:, :, :, :, 1]
    w5 = jnp.stack([qw[:, :, 0, 0], qw[:, :, 1, 1], qw[:, :, 1, 2],
                    qw[:, :, 2, 2], qw[:, :, 3, 1]], axis=0)
    w5 = jnp.transpose(w5, (0, 2, 1))
    bias2 = bias.reshape(o, 1)

    out = pl.pallas_call(
        _body,
        grid=(th // _TI,),
        in_specs=[
            pl.BlockSpec((b, c, h // 2, 2, w), lambda k: (0, 0, 0, 0, 0)),
            pl.BlockSpec((5, c, o), lambda k: (0, 0, 0)),
            pl.BlockSpec((o, 1), lambda k: (0, 0)),
        ],
        out_specs=pl.BlockSpec((b, o, _TI, 2, 2 * tw), lambda k: (0, 0, k, 0, 0)),
        out_shape=jax.ShapeDtypeStruct((b, o, th, 2, 2 * tw), jnp.float32),
        compiler_params=pltpu.CompilerParams(dimension_semantics=("parallel",)),
    )(x5, w5, bias2)

    return out.reshape(b, o, 2 * th, 2 * tw)
